# baseline (speedup 1.0000x reference)
import dataclasses
import functools

import jax
import jax.numpy as jnp
from jax import lax
from jax.experimental import pallas as pl
from jax.experimental.pallas import tpu as pltpu
from jax.experimental.pallas import tpu_sc as plsc

NUM_HEADS = 16
NUM_BINS = 32
MAX_DIST = 1000000.0
T = 2048
BI = 128


def _bins_body(pos_q_ref, pos_k_ref, out_ref):
    q = pos_q_ref[0, :]
    k = pos_k_ref[0, :]
    d = jnp.abs(q[:, None] - k[None, :])
    d = jnp.clip(d, 0.0, MAX_DIST)
    d = jnp.log1p(d)
    dmax = jnp.log1p(jnp.float32(MAX_DIST))
    out_ref[...] = (d / dmax * (NUM_BINS - 1)).astype(jnp.int32)


def _tc_bins(pos):
    return pl.pallas_call(
        _bins_body,
        grid=(T // BI,),
        in_specs=[
            pl.BlockSpec((1, BI), lambda i: (0, i)),
            pl.BlockSpec((1, T), lambda i: (0, 0)),
        ],
        out_specs=pl.BlockSpec((BI, T), lambda i: (i, 0)),
        out_shape=jax.ShapeDtypeStruct((T, T), jnp.int32),
        compiler_params=pltpu.CompilerParams(
            dimension_semantics=("parallel",),
        ),
    )(pos, pos)


def _sc_gather(bins, ptab):
    info = plsc.get_sparse_core_info()
    nc, ns = info.num_cores, info.num_subcores
    nw = nc * ns
    rows_per_w = T // nw
    mesh = plsc.VectorSubcoreMesh(core_axis_name="c", subcore_axis_name="s")
    cp = pltpu.CompilerParams()
    if "needs_layout_passes" in pltpu.CompilerParams.__dataclass_fields__:
        cp = dataclasses.replace(cp, needs_layout_passes=False)

    @functools.partial(
        pl.kernel,
        mesh=mesh,
        compiler_params=cp,
        out_type=jax.ShapeDtypeStruct((NUM_HEADS, T, T), jnp.float32),
        scratch_types=[
            pltpu.VMEM((NUM_HEADS // 2, NUM_BINS), jnp.int32),
            pltpu.VMEM((T,), jnp.int32),
            pltpu.VMEM((NUM_HEADS, T), jnp.float32),
        ],
    )
    def k(bins_hbm, ptab_hbm, out_hbm, tab_v, idx_v, stage_v):
        wid = lax.axis_index("s") * nc + lax.axis_index("c")
        base = wid * rows_per_w
        pltpu.sync_copy(ptab_hbm, tab_v)

        def do_row(r, _):
            row = base + r
            pltpu.sync_copy(bins_hbm.at[row], idx_v)

            def do_vec(v, __):
                bvec = idx_v[pl.ds(v * 16, 16)]
                for p in range(NUM_HEADS // 2):
                    g = plsc.load_gather(tab_v.at[p], [bvec])
                    lo = plsc.bitcast(g << 16, jnp.float32)
                    hi = plsc.bitcast(g & jnp.int32(-65536), jnp.float32)
                    stage_v[2 * p, pl.ds(v * 16, 16)] = lo
                    stage_v[2 * p + 1, pl.ds(v * 16, 16)] = hi
                return __

            lax.fori_loop(0, T // 16, do_vec, 0)
            for h in range(NUM_HEADS):
                pltpu.sync_copy(stage_v.at[h], out_hbm.at[h, row])
            return _

        lax.fori_loop(0, rows_per_w, do_row, 0)

    return k(bins, ptab)


@jax.jit
def kernel(pos, bias):
    b16 = jax.lax.bitcast_convert_type(bias.astype(jnp.bfloat16),
                                       jnp.uint16).astype(jnp.uint32)
    ptab = (b16[0::2, :] | (b16[1::2, :] << 16)).astype(jnp.int32)
    bins = _tc_bins(pos)
    out = _sc_gather(bins, ptab)
    return out.reshape(1, NUM_HEADS, T, T)


# speedup vs baseline: 6.9012x; 6.9012x over previous
import jax
import jax.numpy as jnp
from jax.experimental import pallas as pl
from jax.experimental.pallas import tpu as pltpu

NUM_HEADS = 16
NUM_BINS = 32
MAX_DIST = 1000000.0
T = 2048
BI = 128
JC = 128


def _body(pos_q_ref, pos_k_ref, packed_ref, out_ref):
    q = pos_q_ref[0, :]
    dmax = jnp.log1p(jnp.float32(MAX_DIST))
    tabs = [
        jnp.broadcast_to(packed_ref[p, :][None, :], (8, NUM_BINS))
        for p in range(NUM_HEADS // 2)
    ]
    for j0 in range(0, T, JC):
        k = pos_k_ref[0, j0:j0 + JC]
        d = jnp.abs(q[:, None] - k[None, :])
        d = jnp.clip(d, 0.0, MAX_DIST)
        d = jnp.log1p(d)
        bins = (d / dmax * (NUM_BINS - 1)).astype(jnp.int32)
        for r in range(0, BI, 8):
            br = bins[r:r + 8, :]
            for p in range(NUM_HEADS // 2):
                g = jnp.take_along_axis(tabs[p], br, axis=-1)
                gu = g.astype(jnp.uint32)
                lo = jax.lax.bitcast_convert_type(gu << 16, jnp.float32)
                hi = jax.lax.bitcast_convert_type(gu & jnp.uint32(0xFFFF0000),
                                                  jnp.float32)
                out_ref[0, 2 * p, r:r + 8, j0:j0 + JC] = lo
                out_ref[0, 2 * p + 1, r:r + 8, j0:j0 + JC] = hi


def _call(pos_q_row, pos_row, packed):
    rows = pos_q_row.shape[1]
    return pl.pallas_call(
        _body,
        grid=(rows // BI,),
        in_specs=[
            pl.BlockSpec((1, BI), lambda i: (0, i)),
            pl.BlockSpec((1, T), lambda i: (0, 0)),
            pl.BlockSpec((NUM_HEADS // 2, NUM_BINS), lambda i: (0, 0)),
        ],
        out_specs=pl.BlockSpec((1, NUM_HEADS, BI, T), lambda i: (0, 0, i, 0)),
        out_shape=jax.ShapeDtypeStruct((1, NUM_HEADS, rows, T), jnp.float32),
        compiler_params=pltpu.CompilerParams(
            dimension_semantics=("parallel",),
        ),
    )(pos_q_row, pos_row, packed)


@jax.jit
def kernel(pos, bias):
    b16 = jax.lax.bitcast_convert_type(bias.astype(jnp.bfloat16),
                                       jnp.uint16).astype(jnp.uint32)
    packed = (b16[0::2, :] | (b16[1::2, :] << 16)).astype(jnp.int32)
    return _call(pos, pos, packed)
